# baseline (speedup 1.0000x reference)
import functools

import jax
import jax.numpy as jnp
from jax import lax
from jax.experimental import pallas as pl
from jax.experimental.pallas import tpu as pltpu
from jax.experimental.pallas import tpu_sc as plsc

NW = 32
LANES = 16
NBUF = 4
SPG = 128


def _wid():
    nc = plsc.get_sparse_core_info().num_cores
    return lax.axis_index("s") * nc + lax.axis_index("c")


def _relayout_kernel(V, D, wt_hbm, tailc_hbm, wc_hbm, ib0, ib1, ob0, ob1,
                     isem0, isem1, osem0, osem1):
    WPP = 256
    NP = (V // 128) // 2
    PPW = -(-NP // NW)
    wid = _wid()

    ibufs = (ib0, ib1)
    obufs = (ob0, ob1)
    isems = (isem0, isem1)
    osems = (osem0, osem1)

    lo = wid * PPW
    hi = jnp.minimum(lo + PPW, NP)

    def start_in(p, b):
        for tf in range(D // 8):
            for tw in range(2):
                pltpu.async_copy(
                    wt_hbm.at[pl.ds(8 * tf, 8), pl.ds(128 * (2 * p + tw), 128)],
                    ibufs[b].at[pl.ds(8 * tf, 8), pl.ds(128 * tw, 128)],
                    isems[b],
                )

    def wait_in(b):
        for tf in range(D // 8):
            for tw in range(2):
                pltpu.make_async_copy(
                    wt_hbm.at[pl.ds(0, 8), pl.ds(0, 128)],
                    ibufs[b].at[pl.ds(0, 8), pl.ds(0, 128)],
                    isems[b],
                ).wait()

    rowvecs = [
        jax.lax.iota(jnp.int32, LANES) + k * LANES for k in range(D // LANES)
    ]

    def transpose(b):
        ibuf = ibufs[b]
        obuf = obufs[b]

        def tbody(m, carry):
            for half in range(2):
                c = 2 * m + half
                colv = jnp.full((LANES,), c, jnp.int32)
                for k in range(D // LANES):
                    v = plsc.load_gather(ibuf, [rowvecs[k], colv])
                    obuf[m, pl.ds(half * D + k * LANES, LANES)] = v
            return carry

        lax.fori_loop(0, WPP // 2, tbody, 0)

    def start_out(p, b):
        pltpu.async_copy(obufs[b], wc_hbm.at[pl.ds(128 * p, 128)], osems[b])

    def wait_out(b):
        pltpu.make_async_copy(
            obufs[b], wc_hbm.at[pl.ds(0, 128)], osems[b]
        ).wait()

    @pl.when(lo < hi)
    def _():
        start_in(lo, 0)

    @pl.when(lo + 1 < hi)
    def _():
        start_in(lo + 1, 1)

    def body(g, carry):
        for b in range(2):
            p = lo + 2 * g + b

            @pl.when(p < hi)
            def _():
                wait_in(b)

                @pl.when(2 * g + b >= 2)
                def _():
                    wait_out(b)

                transpose(b)

                @pl.when(p + 2 < hi)
                def _():
                    start_in(p + 2, b)

                start_out(p, b)
        return carry

    lax.fori_loop(0, PPW // 2 + 1, body, 0)

    n = jnp.maximum(hi - lo, 0)

    @pl.when(n >= 1)
    def _():
        wait_out(0)

    @pl.when(n >= 2)
    def _():
        wait_out(1)

    tail = V - WPP * NP
    if tail:

        @pl.when(wid == 0)
        def _():
            pltpu.sync_copy(tailc_hbm, ob0.at[pl.ds(0, tail // 2)])
            pltpu.sync_copy(
                ob0.at[pl.ds(0, tail // 2)],
                wc_hbm.at[pl.ds(128 * NP, tail // 2)],
            )


def _emb_mean_kernel(B, L, D, idx_hbm, table_hbm, out_hbm,
                     idx_v, rows0, rows1, rows2, rows3, acc_v,
                     sem0, sem1, sem2, sem3):
    BPW = B // NW
    NBLK = BPW // SPG
    NV = D // LANES
    NS = L * NBLK
    inv = jnp.float32(1.0 / L)
    wid = _wid()

    pltpu.sync_copy(idx_hbm.at[:, pl.ds(wid * BPW, BPW)], idx_v)

    def zbody(r, carry):
        for k in range(NV):
            acc_v[r, pl.ds(k * LANES, LANES)] = jnp.zeros((LANES,), jnp.float32)
        return carry

    lax.fori_loop(0, BPW, zbody, 0)

    bufs = (rows0, rows1, rows2, rows3)
    sems = (sem0, sem1, sem2, sem3)

    def start(s, b):
        l = s // NBLK
        blk = s - l * NBLK
        idx_slice = idx_v.at[l, pl.ds(blk * SPG, SPG)]
        pltpu.async_copy(table_hbm.at[idx_slice], bufs[b], sems[b])

    def wait(b):
        pltpu.make_async_copy(
            table_hbm.at[idx_v.at[0, pl.ds(0, SPG)]], bufs[b], sems[b]
        ).wait()

    def accumulate(s, b):
        blk = s - (s // NBLK) * NBLK
        base = blk * SPG
        rows = bufs[b]

        def abody(r, carry):
            for k in range(NV):
                sl = pl.ds(k * LANES, LANES)
                plsc.addupdate(acc_v.at[base + r, sl], rows[r, sl])
            return carry

        lax.fori_loop(0, SPG, abody, 0)

    for b in range(NBUF):
        start(b, b)

    def body(g, carry):
        for b in range(NBUF):
            s = NBUF * g + b
            wait(b)
            accumulate(s, b)

            @pl.when(s + NBUF < NS)
            def _():
                start(s + NBUF, b)
        return carry

    lax.fori_loop(0, NS // NBUF, body, 0)

    def sbody(r, carry):
        for k in range(NV):
            sl = pl.ds(k * LANES, LANES)
            acc_v[r, sl] = acc_v[r, sl] * inv
        return carry

    lax.fori_loop(0, BPW, sbody, 0)
    pltpu.sync_copy(acc_v, out_hbm.at[pl.ds(wid * BPW, BPW)])


@functools.partial(jax.jit, static_argnames=("B", "L", "V", "D"))
def _emb_mean(idx_t, Wt, tail_c, B, L, V, D):
    BPW = B // NW
    mesh = plsc.VectorSubcoreMesh(core_axis_name="c", subcore_axis_name="s")
    WPP = 256

    wc = pl.kernel(
        functools.partial(_relayout_kernel, V, D),
        out_type=jax.ShapeDtypeStruct((V // 2, 2 * D), jnp.float32),
        mesh=mesh,
        compiler_params=pltpu.CompilerParams(
            use_tc_tiling_on_sc=True, needs_layout_passes=False
        ),
        scratch_types=[
            pltpu.VMEM((D, WPP), jnp.float32),
            pltpu.VMEM((D, WPP), jnp.float32),
            pltpu.VMEM((WPP // 2, 2 * D), jnp.float32),
            pltpu.VMEM((WPP // 2, 2 * D), jnp.float32),
            pltpu.SemaphoreType.DMA,
            pltpu.SemaphoreType.DMA,
            pltpu.SemaphoreType.DMA,
            pltpu.SemaphoreType.DMA,
        ],
    )(Wt, tail_c)

    table = wc.reshape(V, D)

    return pl.kernel(
        functools.partial(_emb_mean_kernel, B, L, D),
        out_type=jax.ShapeDtypeStruct((B, D), jnp.float32),
        mesh=mesh,
        compiler_params=pltpu.CompilerParams(use_tc_tiling_on_sc=False),
        scratch_types=[
            pltpu.VMEM((L, BPW), jnp.int32),
            pltpu.VMEM((SPG, D), jnp.float32),
            pltpu.VMEM((SPG, D), jnp.float32),
            pltpu.VMEM((SPG, D), jnp.float32),
            pltpu.VMEM((SPG, D), jnp.float32),
            pltpu.VMEM((BPW, D), jnp.float32),
            pltpu.SemaphoreType.DMA,
            pltpu.SemaphoreType.DMA,
            pltpu.SemaphoreType.DMA,
            pltpu.SemaphoreType.DMA,
        ],
    )(idx_t, table)


def kernel(word_ids, W):
    B, L = word_ids.shape
    V, D = W.shape
    BPW = B // NW
    assert B % NW == 0 and BPW % SPG == 0 and D % LANES == 0
    idx_t = word_ids.astype(jnp.int32).T
    Wt = W.T
    ntail = V % 256
    tail_c = W[V - ntail:, :].reshape(ntail // 2, 2 * D)
    return _emb_mean(idx_t, Wt, tail_c, B, L, V, D)


# speedup vs baseline: 2.2161x; 2.2161x over previous
import functools

import jax
import jax.numpy as jnp
from jax import lax
from jax.experimental import pallas as pl
from jax.experimental.pallas import tpu as pltpu
from jax.experimental.pallas import tpu_sc as plsc

NW = 32
LANES = 16
NBUF = 4
SPG = 128


def _emb_mean_kernel(B, L, D, idx_hbm, table_hbm, out_hbm,
                     idx_v, rows0, rows1, rows2, rows3, acc_v,
                     sem0, sem1, sem2, sem3):
    BPW = B // NW
    NBLK = BPW // SPG
    NV = D // LANES
    NS = L * NBLK
    inv = jnp.float32(1.0 / L)

    nc = plsc.get_sparse_core_info().num_cores
    wid = lax.axis_index("s") * nc + lax.axis_index("c")

    pltpu.sync_copy(idx_hbm.at[:, pl.ds(wid * BPW, BPW)], idx_v)

    def zbody(r, carry):
        for k in range(NV):
            acc_v[r, pl.ds(k * LANES, LANES)] = jnp.zeros((LANES,), jnp.float32)
        return carry

    lax.fori_loop(0, BPW, zbody, 0)

    bufs = (rows0, rows1, rows2, rows3)
    sems = (sem0, sem1, sem2, sem3)

    def start(s, b):
        l = s // NBLK
        blk = s - l * NBLK
        idx_slice = idx_v.at[l, pl.ds(blk * SPG, SPG)]
        pltpu.async_copy(table_hbm.at[idx_slice], bufs[b], sems[b])

    def wait(b):
        pltpu.make_async_copy(
            table_hbm.at[idx_v.at[0, pl.ds(0, SPG)]], bufs[b], sems[b]
        ).wait()

    def accumulate(s, b):
        blk = s - (s // NBLK) * NBLK
        base = blk * SPG
        rows = bufs[b]

        def abody(r, carry):
            for k in range(NV):
                sl = pl.ds(k * LANES, LANES)
                plsc.addupdate(acc_v.at[base + r, sl], rows[r, sl])
            return carry

        lax.fori_loop(0, SPG, abody, 0)

    for b in range(NBUF):
        start(b, b)

    def body(g, carry):
        for b in range(NBUF):
            s = NBUF * g + b
            wait(b)
            accumulate(s, b)

            @pl.when(s + NBUF < NS)
            def _():
                start(s + NBUF, b)
        return carry

    lax.fori_loop(0, NS // NBUF, body, 0)

    def sbody(r, carry):
        for k in range(NV):
            sl = pl.ds(k * LANES, LANES)
            acc_v[r, sl] = acc_v[r, sl] * inv
        return carry

    lax.fori_loop(0, BPW, sbody, 0)
    pltpu.sync_copy(acc_v, out_hbm.at[pl.ds(wid * BPW, BPW)])


@functools.partial(jax.jit, static_argnames=("B", "L", "D"))
def _emb_mean(idx_t, W, B, L, D):
    BPW = B // NW
    mesh = plsc.VectorSubcoreMesh(core_axis_name="c", subcore_axis_name="s")
    return pl.kernel(
        functools.partial(_emb_mean_kernel, B, L, D),
        out_type=jax.ShapeDtypeStruct((B, D), jnp.float32),
        mesh=mesh,
        compiler_params=pltpu.CompilerParams(use_tc_tiling_on_sc=False),
        scratch_types=[
            pltpu.VMEM((L, BPW), jnp.int32),
            pltpu.VMEM((SPG, D), jnp.float32),
            pltpu.VMEM((SPG, D), jnp.float32),
            pltpu.VMEM((SPG, D), jnp.float32),
            pltpu.VMEM((SPG, D), jnp.float32),
            pltpu.VMEM((BPW, D), jnp.float32),
            pltpu.SemaphoreType.DMA,
            pltpu.SemaphoreType.DMA,
            pltpu.SemaphoreType.DMA,
            pltpu.SemaphoreType.DMA,
        ],
    )(idx_t, W)


def kernel(word_ids, W):
    B, L = word_ids.shape
    D = W.shape[1]
    BPW = B // NW
    assert B % NW == 0 and BPW % SPG == 0 and D % LANES == 0
    idx_t = word_ids.astype(jnp.int32).T
    return _emb_mean(idx_t, W, B, L, D)
